# baseline (speedup 1.0000x reference)
import jax
import jax.numpy as jnp
from jax import lax
from jax.experimental import pallas as pl
from jax.experimental.pallas import tpu as pltpu
from jax.experimental.pallas import tpu_sc as plsc

N = 160000
D = 256
NUM_REL = 64

N_SC = 38400
N_TC = N - N_SC

NC = 2
NS = 16
NW = NC * NS
ROWS_PER_W = N_SC // NW
CHUNK = 40
NCHUNK = ROWS_PER_W // CHUNK
NBUF = 2
LANES = 16
DSTEPS = D // LANES

BN = 8192


def _sc_body(src_hbm, dst_hbm, ridx_hbm, emb_hbm, scale_hbm, out_hbm,
             idx_v, scal_v, rvec, src_buf, dst_buf, out_v, tile_v, sems, rsem):
    wid = lax.axis_index("s") * NC + lax.axis_index("c")
    base = N_TC + wid * ROWS_PER_W

    pltpu.sync_copy(ridx_hbm, idx_v)
    pltpu.sync_copy(scale_hbm, scal_v)
    pltpu.async_copy(emb_hbm.at[idx_v], rvec, rsem).wait()
    s_vec = scal_v[pl.ds(0, LANES)]
    for j in range(DSTEPS):
        rvec[0, pl.ds(j * LANES, LANES)] = rvec[0, pl.ds(j * LANES, LANES)] * s_vec
    r_regs = [rvec[0, pl.ds(j * LANES, LANES)] for j in range(DSTEPS)]

    def start(c, b):
        rows = pl.ds(base + c * CHUNK, CHUNK)
        pltpu.make_async_copy(src_hbm.at[rows], src_buf.at[b], sems.at[b]).start()
        pltpu.make_async_copy(dst_hbm.at[rows], dst_buf.at[b], sems.at[b]).start()

    def wait(c, b):
        rows = pl.ds(base + c * CHUNK, CHUNK)
        pltpu.make_async_copy(src_hbm.at[rows], src_buf.at[b], sems.at[b]).wait()
        pltpu.make_async_copy(dst_hbm.at[rows], dst_buf.at[b], sems.at[b]).wait()

    for b in range(NBUF):
        start(b, b)

    lane_ids = lax.iota(jnp.int32, LANES)

    def row_acc(sbuf, dbuf, i):
        acc = sbuf[i, pl.ds(0, LANES)] * r_regs[0] * dbuf[i, pl.ds(0, LANES)]
        for j in range(1, DSTEPS):
            sl = pl.ds(j * LANES, LANES)
            acc = acc + sbuf[i, sl] * r_regs[j] * dbuf[i, sl]
        return acc

    def tile_row_sums(tile):
        sv = None
        for j in range(DSTEPS):
            col = plsc.load_gather(
                tile, [lane_ids, jnp.full((LANES,), j, jnp.int32)])
            sv = col if sv is None else sv + col
        return sv

    NFULL = CHUNK // LANES
    NREM = CHUNK % LANES

    def compute_chunk(c, b):
        sbuf = src_buf.at[b]
        dbuf = dst_buf.at[b]

        def group_body(g, _):
            ro = g * LANES
            for k in range(LANES):
                tile_v[k, pl.ds(0, LANES)] = row_acc(sbuf, dbuf, ro + k)
            sv = tile_row_sums(tile_v)
            plsc.store_scatter(out_v, [lane_ids + c * CHUNK + ro], sv)
            return 0

        lax.fori_loop(0, NFULL, group_body, 0)

        if NREM:
            ro = NFULL * LANES
            for k in range(NREM):
                tile_v[k, pl.ds(0, LANES)] = row_acc(sbuf, dbuf, ro + k)
            sv = tile_row_sums(tile_v)
            plsc.store_scatter(out_v, [lane_ids + c * CHUNK + ro], sv,
                               mask=lane_ids < NREM)

    def chunk_pair(it, carry):
        c0 = it * NBUF
        for b in range(NBUF):
            c = c0 + b
            wait(c, b)
            compute_chunk(c, b)

            @pl.when(c + NBUF < NCHUNK)
            def _(c=c, b=b):
                start(c + NBUF, b)
        return carry

    lax.fori_loop(0, NCHUNK // NBUF, chunk_pair, 0)
    for c in range(NCHUNK - NCHUNK % NBUF, NCHUNK):
        b = c % NBUF
        wait(c, b)
        compute_chunk(c, b)
    pltpu.sync_copy(out_v, out_hbm.at[pl.ds(wid * ROWS_PER_W, ROWS_PER_W)])


def _sc_kernel(z_src, z_dst, ridx, rel_emb, scale16):
    mesh = plsc.VectorSubcoreMesh(
        core_axis_name="c", subcore_axis_name="s",
        num_cores=NC, num_subcores=NS,
    )
    f = pl.kernel(
        _sc_body,
        out_type=jax.ShapeDtypeStruct((N_SC,), jnp.float32),
        mesh=mesh,
        scratch_types=[
            pltpu.VMEM((1,), jnp.int32),
            pltpu.VMEM((LANES,), jnp.float32),
            pltpu.VMEM((1, D), jnp.float32),
            pltpu.VMEM((NBUF, CHUNK, D), jnp.float32),
            pltpu.VMEM((NBUF, CHUNK, D), jnp.float32),
            pltpu.VMEM((ROWS_PER_W,), jnp.float32),
            pltpu.VMEM((LANES, LANES), jnp.float32),
            pltpu.SemaphoreType.DMA((NBUF,)),
            pltpu.SemaphoreType.DMA,
        ],
        compiler_params=pltpu.CompilerParams(needs_layout_passes=False),
    )
    return f(z_src, z_dst, ridx, rel_emb, scale16)


def _tc_body(idx_ref, scale_ref, emb_ref, src_ref, dst_ref, out_ref):
    r = emb_ref[idx_ref[0], :] * scale_ref[0]
    t = src_ref[:] * dst_ref[:]
    out_ref[:] = jax.lax.dot_general(
        t, r, (((1,), (0,)), ((), ())),
        preferred_element_type=jnp.float32)


def _tc_kernel(z_src, z_dst, ridx, rel_emb, scale):
    grid = ((N_TC + BN - 1) // BN,)
    return pl.pallas_call(
        _tc_body,
        grid=grid,
        in_specs=[
            pl.BlockSpec(memory_space=pltpu.SMEM),
            pl.BlockSpec(memory_space=pltpu.SMEM),
            pl.BlockSpec((NUM_REL, D), lambda i: (0, 0)),
            pl.BlockSpec((BN, D), lambda i: (i, 0)),
            pl.BlockSpec((BN, D), lambda i: (i, 0)),
        ],
        out_specs=pl.BlockSpec((BN,), lambda i: (i,)),
        out_shape=jax.ShapeDtypeStruct((N_TC,), jnp.float32),
    )(ridx, scale, rel_emb, z_src, z_dst)


def _tc_kernel_full(z_src, z_dst, ridx, rel_emb, scale):
    grid = ((N + BN - 1) // BN,)
    return pl.pallas_call(
        _tc_body,
        grid=grid,
        in_specs=[
            pl.BlockSpec(memory_space=pltpu.SMEM),
            pl.BlockSpec(memory_space=pltpu.SMEM),
            pl.BlockSpec((NUM_REL, D), lambda i: (0, 0)),
            pl.BlockSpec((BN, D), lambda i: (i, 0)),
            pl.BlockSpec((BN, D), lambda i: (i, 0)),
        ],
        out_specs=pl.BlockSpec((BN,), lambda i: (i,)),
        out_shape=jax.ShapeDtypeStruct((N,), jnp.float32),
        compiler_params=pltpu.CompilerParams(
            dimension_semantics=("arbitrary",)),
    )(ridx, scale, rel_emb, z_src, z_dst)


CH = 3200
NCH = N // CH
NBT = 5


def _tcm_body(idx_ref, scale_ref, emb_ref, src_hbm, dst_hbm, out_hbm,
              sbuf, dbuf, *rest):
    obufs = list(rest[:NBT])
    insem, outsem = rest[NBT], rest[NBT + 1]
    r = emb_ref[idx_ref[0], :] * scale_ref[0]

    def start_in(c, b):
        rows = pl.ds(c * CH, CH)
        pltpu.make_async_copy(src_hbm.at[rows], sbuf.at[b], insem.at[b]).start()
        pltpu.make_async_copy(dst_hbm.at[rows], dbuf.at[b], insem.at[b]).start()

    def wait_in(c, b):
        rows = pl.ds(c * CH, CH)
        pltpu.make_async_copy(src_hbm.at[rows], sbuf.at[b], insem.at[b]).wait()
        pltpu.make_async_copy(dst_hbm.at[rows], dbuf.at[b], insem.at[b]).wait()

    for b in range(NBT):
        start_in(b, b)

    def step(c, b):
        @pl.when(c >= NBT)
        def _():
            pltpu.make_async_copy(
                obufs[b], out_hbm.at[pl.ds((c - NBT) * CH, CH)],
                outsem.at[b]).wait()

        wait_in(c, b)
        obufs[b][:] = jnp.sum(sbuf[b] * dbuf[b] * r[None, :], axis=1)
        pltpu.make_async_copy(
            obufs[b], out_hbm.at[pl.ds(c * CH, CH)], outsem.at[b]).start()

        @pl.when(c + NBT < NCH)
        def _():
            start_in(c + NBT, b)

    def group(g, carry):
        for b in range(NBT):
            step(g * NBT + b, b)
        return carry

    lax.fori_loop(0, NCH // NBT, group, 0, unroll=False)

    for b in range(NBT):
        pltpu.make_async_copy(
            obufs[b], out_hbm.at[pl.ds((NCH - NBT + b) * CH, CH)],
            outsem.at[b]).wait()


def _tcm_kernel(z_src, z_dst, ridx, rel_emb, scale):
    return pl.pallas_call(
        _tcm_body,
        in_specs=[
            pl.BlockSpec(memory_space=pltpu.SMEM),
            pl.BlockSpec(memory_space=pltpu.SMEM),
            pl.BlockSpec(memory_space=pltpu.VMEM),
            pl.BlockSpec(memory_space=pl.ANY),
            pl.BlockSpec(memory_space=pl.ANY),
        ],
        out_specs=pl.BlockSpec(memory_space=pl.ANY),
        out_shape=jax.ShapeDtypeStruct((N,), jnp.float32),
        scratch_shapes=(
            [pltpu.VMEM((NBT, CH, D), jnp.float32),
             pltpu.VMEM((NBT, CH, D), jnp.float32)]
            + [pltpu.VMEM((CH,), jnp.float32) for _ in range(NBT)]
            + [pltpu.SemaphoreType.DMA((NBT,)),
               pltpu.SemaphoreType.DMA((NBT,))]
        ),
    )(ridx, scale, rel_emb, z_src, z_dst)


def kernel(z_src, z_dst, rel_idx, rel_emb, score_scale):
    ridx = jnp.asarray(rel_idx, jnp.int32).reshape((1,))
    scale1 = jnp.asarray(score_scale, jnp.float32).reshape((1,))
    return _tcm_kernel(z_src, z_dst, ridx, rel_emb, scale1)


# speedup vs baseline: 1.0897x; 1.0897x over previous
import jax
import jax.numpy as jnp
from jax import lax
from jax.experimental import pallas as pl
from jax.experimental.pallas import tpu as pltpu
from jax.experimental.pallas import tpu_sc as plsc

N = 160000
D = 256
NUM_REL = 64

N_SC = 38400
N_TC = N - N_SC

NC = 2
NS = 16
NW = NC * NS
ROWS_PER_W = N_SC // NW
CHUNK = 40
NCHUNK = ROWS_PER_W // CHUNK
NBUF = 2
LANES = 16
DSTEPS = D // LANES

BN = 8192


def _sc_body(src_hbm, dst_hbm, ridx_hbm, emb_hbm, scale_hbm, out_hbm,
             idx_v, scal_v, rvec, src_buf, dst_buf, out_v, tile_v, sems, rsem):
    wid = lax.axis_index("s") * NC + lax.axis_index("c")
    base = N_TC + wid * ROWS_PER_W

    pltpu.sync_copy(ridx_hbm, idx_v)
    pltpu.sync_copy(scale_hbm, scal_v)
    pltpu.async_copy(emb_hbm.at[idx_v], rvec, rsem).wait()
    s_vec = scal_v[pl.ds(0, LANES)]
    for j in range(DSTEPS):
        rvec[0, pl.ds(j * LANES, LANES)] = rvec[0, pl.ds(j * LANES, LANES)] * s_vec
    r_regs = [rvec[0, pl.ds(j * LANES, LANES)] for j in range(DSTEPS)]

    def start(c, b):
        rows = pl.ds(base + c * CHUNK, CHUNK)
        pltpu.make_async_copy(src_hbm.at[rows], src_buf.at[b], sems.at[b]).start()
        pltpu.make_async_copy(dst_hbm.at[rows], dst_buf.at[b], sems.at[b]).start()

    def wait(c, b):
        rows = pl.ds(base + c * CHUNK, CHUNK)
        pltpu.make_async_copy(src_hbm.at[rows], src_buf.at[b], sems.at[b]).wait()
        pltpu.make_async_copy(dst_hbm.at[rows], dst_buf.at[b], sems.at[b]).wait()

    for b in range(NBUF):
        start(b, b)

    lane_ids = lax.iota(jnp.int32, LANES)

    def row_acc(sbuf, dbuf, i):
        acc = sbuf[i, pl.ds(0, LANES)] * r_regs[0] * dbuf[i, pl.ds(0, LANES)]
        for j in range(1, DSTEPS):
            sl = pl.ds(j * LANES, LANES)
            acc = acc + sbuf[i, sl] * r_regs[j] * dbuf[i, sl]
        return acc

    def tile_row_sums(tile):
        sv = None
        for j in range(DSTEPS):
            col = plsc.load_gather(
                tile, [lane_ids, jnp.full((LANES,), j, jnp.int32)])
            sv = col if sv is None else sv + col
        return sv

    NFULL = CHUNK // LANES
    NREM = CHUNK % LANES

    def compute_chunk(c, b):
        sbuf = src_buf.at[b]
        dbuf = dst_buf.at[b]

        def group_body(g, _):
            ro = g * LANES
            for k in range(LANES):
                tile_v[k, pl.ds(0, LANES)] = row_acc(sbuf, dbuf, ro + k)
            sv = tile_row_sums(tile_v)
            plsc.store_scatter(out_v, [lane_ids + c * CHUNK + ro], sv)
            return 0

        lax.fori_loop(0, NFULL, group_body, 0)

        if NREM:
            ro = NFULL * LANES
            for k in range(NREM):
                tile_v[k, pl.ds(0, LANES)] = row_acc(sbuf, dbuf, ro + k)
            sv = tile_row_sums(tile_v)
            plsc.store_scatter(out_v, [lane_ids + c * CHUNK + ro], sv,
                               mask=lane_ids < NREM)

    def chunk_pair(it, carry):
        c0 = it * NBUF
        for b in range(NBUF):
            c = c0 + b
            wait(c, b)
            compute_chunk(c, b)

            @pl.when(c + NBUF < NCHUNK)
            def _(c=c, b=b):
                start(c + NBUF, b)
        return carry

    lax.fori_loop(0, NCHUNK // NBUF, chunk_pair, 0)
    for c in range(NCHUNK - NCHUNK % NBUF, NCHUNK):
        b = c % NBUF
        wait(c, b)
        compute_chunk(c, b)
    pltpu.sync_copy(out_v, out_hbm.at[pl.ds(wid * ROWS_PER_W, ROWS_PER_W)])


def _sc_kernel(z_src, z_dst, ridx, rel_emb, scale16):
    mesh = plsc.VectorSubcoreMesh(
        core_axis_name="c", subcore_axis_name="s",
        num_cores=NC, num_subcores=NS,
    )
    f = pl.kernel(
        _sc_body,
        out_type=jax.ShapeDtypeStruct((N_SC,), jnp.float32),
        mesh=mesh,
        scratch_types=[
            pltpu.VMEM((1,), jnp.int32),
            pltpu.VMEM((LANES,), jnp.float32),
            pltpu.VMEM((1, D), jnp.float32),
            pltpu.VMEM((NBUF, CHUNK, D), jnp.float32),
            pltpu.VMEM((NBUF, CHUNK, D), jnp.float32),
            pltpu.VMEM((ROWS_PER_W,), jnp.float32),
            pltpu.VMEM((LANES, LANES), jnp.float32),
            pltpu.SemaphoreType.DMA((NBUF,)),
            pltpu.SemaphoreType.DMA,
        ],
        compiler_params=pltpu.CompilerParams(needs_layout_passes=False),
    )
    return f(z_src, z_dst, ridx, rel_emb, scale16)


def _tc_body(idx_ref, scale_ref, emb_ref, src_ref, dst_ref, out_ref):
    r = emb_ref[idx_ref[0], :] * scale_ref[0]
    t = src_ref[:] * dst_ref[:]
    out_ref[:] = jax.lax.dot_general(
        t, r, (((1,), (0,)), ((), ())),
        preferred_element_type=jnp.float32)


def _tc_kernel(z_src, z_dst, ridx, rel_emb, scale):
    grid = ((N_TC + BN - 1) // BN,)
    return pl.pallas_call(
        _tc_body,
        grid=grid,
        in_specs=[
            pl.BlockSpec(memory_space=pltpu.SMEM),
            pl.BlockSpec(memory_space=pltpu.SMEM),
            pl.BlockSpec((NUM_REL, D), lambda i: (0, 0)),
            pl.BlockSpec((BN, D), lambda i: (i, 0)),
            pl.BlockSpec((BN, D), lambda i: (i, 0)),
        ],
        out_specs=pl.BlockSpec((BN,), lambda i: (i,)),
        out_shape=jax.ShapeDtypeStruct((N_TC,), jnp.float32),
    )(ridx, scale, rel_emb, z_src, z_dst)


def _tc_kernel_full(z_src, z_dst, ridx, rel_emb, scale):
    grid = ((N + BN - 1) // BN,)
    return pl.pallas_call(
        _tc_body,
        grid=grid,
        in_specs=[
            pl.BlockSpec(memory_space=pltpu.SMEM),
            pl.BlockSpec(memory_space=pltpu.SMEM),
            pl.BlockSpec((NUM_REL, D), lambda i: (0, 0)),
            pl.BlockSpec((BN, D), lambda i: (i, 0)),
            pl.BlockSpec((BN, D), lambda i: (i, 0)),
        ],
        out_specs=pl.BlockSpec((BN,), lambda i: (i,)),
        out_shape=jax.ShapeDtypeStruct((N,), jnp.float32),
        compiler_params=pltpu.CompilerParams(
            dimension_semantics=("arbitrary",)),
    )(ridx, scale, rel_emb, z_src, z_dst)


CH = 1280
NCH = N // CH
NBT = 5


def _tcm_body(idx_ref, scale_ref, emb_ref, src_hbm, dst_hbm, out_hbm,
              sbuf, dbuf, *rest):
    obufs = list(rest[:NBT])
    insem, outsem = rest[NBT], rest[NBT + 1]
    r = emb_ref[idx_ref[0], :] * scale_ref[0]

    def start_in(c, b):
        rows = pl.ds(c * CH, CH)
        pltpu.make_async_copy(src_hbm.at[rows], sbuf.at[b], insem.at[b]).start()
        pltpu.make_async_copy(dst_hbm.at[rows], dbuf.at[b], insem.at[b]).start()

    def wait_in(c, b):
        rows = pl.ds(c * CH, CH)
        pltpu.make_async_copy(src_hbm.at[rows], sbuf.at[b], insem.at[b]).wait()
        pltpu.make_async_copy(dst_hbm.at[rows], dbuf.at[b], insem.at[b]).wait()

    for b in range(NBT):
        start_in(b, b)

    def step(c, b):
        @pl.when(c >= NBT)
        def _():
            pltpu.make_async_copy(
                obufs[b], out_hbm.at[pl.ds((c - NBT) * CH, CH)],
                outsem.at[b]).wait()

        wait_in(c, b)
        obufs[b][:] = jnp.sum(sbuf[b] * dbuf[b] * r[None, :], axis=1)
        pltpu.make_async_copy(
            obufs[b], out_hbm.at[pl.ds(c * CH, CH)], outsem.at[b]).start()

        @pl.when(c + NBT < NCH)
        def _():
            start_in(c + NBT, b)

    def group(g, carry):
        for b in range(NBT):
            step(g * NBT + b, b)
        return carry

    lax.fori_loop(0, NCH // NBT, group, 0, unroll=False)

    for b in range(NBT):
        pltpu.make_async_copy(
            obufs[b], out_hbm.at[pl.ds((NCH - NBT + b) * CH, CH)],
            outsem.at[b]).wait()


def _tcm_kernel(z_src, z_dst, ridx, rel_emb, scale):
    return pl.pallas_call(
        _tcm_body,
        in_specs=[
            pl.BlockSpec(memory_space=pltpu.SMEM),
            pl.BlockSpec(memory_space=pltpu.SMEM),
            pl.BlockSpec(memory_space=pltpu.VMEM),
            pl.BlockSpec(memory_space=pl.ANY),
            pl.BlockSpec(memory_space=pl.ANY),
        ],
        out_specs=pl.BlockSpec(memory_space=pl.ANY),
        out_shape=jax.ShapeDtypeStruct((N,), jnp.float32),
        scratch_shapes=(
            [pltpu.VMEM((NBT, CH, D), jnp.float32),
             pltpu.VMEM((NBT, CH, D), jnp.float32)]
            + [pltpu.VMEM((CH,), jnp.float32) for _ in range(NBT)]
            + [pltpu.SemaphoreType.DMA((NBT,)),
               pltpu.SemaphoreType.DMA((NBT,))]
        ),
    )(ridx, scale, rel_emb, z_src, z_dst)


def kernel(z_src, z_dst, rel_idx, rel_emb, score_scale):
    ridx = jnp.asarray(rel_idx, jnp.int32).reshape((1,))
    scale1 = jnp.asarray(score_scale, jnp.float32).reshape((1,))
    return _tcm_kernel(z_src, z_dst, ridx, rel_emb, scale1)
